# baseline (speedup 1.0000x reference)
import functools

import jax
import jax.numpy as jnp
from jax import lax
from jax.experimental import pallas as pl
from jax.experimental.pallas import tpu as pltpu
from jax.experimental.pallas import tpu_sc as plsc


def _sc_gather(table, idx):
    NQ, D = table.shape
    B = idx.shape[0]
    table3 = table.reshape(NQ // 8, 8, D)
    info = plsc.get_sparse_core_info()
    nc, ns, L = info.num_cores, info.num_subcores, info.num_lanes
    nw = nc * ns
    n = B // nw
    K = 16
    nch = n // K
    mesh = plsc.VectorSubcoreMesh(core_axis_name="c", subcore_axis_name="s")

    @functools.partial(
        pl.kernel,
        mesh=mesh,
        out_type=jax.ShapeDtypeStruct((B, D), jnp.float32),
        scratch_types=[
            pltpu.VMEM((n,), jnp.int32),
            pltpu.VMEM((2, K, 8, D), jnp.float32),
            pltpu.VMEM((n, D), jnp.float32),
            pltpu.SemaphoreType.DMA,
            pltpu.SemaphoreType.DMA,
        ],
        compiler_params=pltpu.CompilerParams(needs_layout_passes=False),
    )
    def k(table_hbm, idx_hbm, out_hbm, idx_v, slabs, rows, sem0, sem1):
        wid = lax.axis_index("s") * nc + lax.axis_index("c")
        base = wid * n
        pltpu.sync_copy(idx_hbm.at[pl.ds(base, n)], idx_v)
        sems = (sem0, sem1)

        def fire(c, bank, sem):
            qv = idx_v[pl.ds(c * K, L)]
            slabv = jnp.right_shift(qv, 3)
            for j in range(K):
                pltpu.async_copy(table_hbm.at[slabv[j]],
                                 slabs.at[bank, j], sem)

        def drain_extract(c, bank, sem):
            for j in range(K):
                pltpu.make_async_copy(table_hbm.at[0], slabs.at[bank, j],
                                      sem).wait()
            qv = idx_v[pl.ds(c * K, L)]
            subv = jnp.bitwise_and(qv, 7)
            r0 = c * K
            for j in range(K):
                sub = subv[j]
                for cc in range(D // L):
                    rows[r0 + j, pl.ds(cc * L, L)] = slabs[bank, j, sub,
                                                           pl.ds(cc * L, L)]

        fire(0, 0, sem0)

        def body(h, _):
            c0 = 2 * h
            fire(c0 + 1, 1, sem1)
            drain_extract(c0, 0, sem0)
            nxt = lax.rem(c0 + 2, nch)
            fire(nxt, 0, sem0)
            drain_extract(c0 + 1, 1, sem1)
            return 0

        lax.fori_loop(0, nch // 2, body, 0)
        for j in range(K):
            pltpu.make_async_copy(table_hbm.at[0], slabs.at[0, j],
                                  sem0).wait()
        pltpu.sync_copy(rows, out_hbm.at[pl.ds(base, n)])

    return k(table3, idx)


def _tc_a(t_ref, df_ref, S_ref, Wd_ref, bd_ref, W1m_ref, W1a_ref, w1p_ref,
          b1_ref, E1_ref, mu_ref, a_ref):
    mask = (t_ref[...] != 0).astype(jnp.float32)
    cnt = jnp.maximum(jnp.sum(mask, axis=1, keepdims=True), 1.0)
    mu = lax.dot_general(mask, S_ref[...], (((1,), (0,)), ((), ())),
                         preferred_element_type=jnp.float32) / cnt
    a = jnp.dot(df_ref[...], Wd_ref[...],
                preferred_element_type=jnp.float32) + bd_ref[...]
    p23 = jnp.sum(mu * a, axis=-1, keepdims=True)
    E1 = (jnp.dot(mu, W1m_ref[...], preferred_element_type=jnp.float32)
          + jnp.dot(a, W1a_ref[...], preferred_element_type=jnp.float32)
          + p23 * w1p_ref[2:3, :] + b1_ref[...])
    E1_ref[...] = E1
    mu_ref[...] = mu
    a_ref[...] = a


def _tc_b(q_ref, E1_ref, mu_ref, a_ref, W1q_ref, w1p_ref, W2_ref, b2_ref,
          e_ref, p_ref):
    q = q_ref[...]
    mu = mu_ref[...]
    a = a_ref[...]
    p12 = jnp.sum(q * mu, axis=-1, keepdims=True)
    p13 = jnp.sum(q * a, axis=-1, keepdims=True)
    z = (E1_ref[...] + jnp.dot(q, W1q_ref[...],
                               preferred_element_type=jnp.float32)
         + p12 * w1p_ref[0:1, :] + p13 * w1p_ref[1:2, :])
    e = jnp.maximum(z, 0.0)
    e_ref[...] = e
    p_ref[...] = jnp.dot(e, W2_ref[...],
                         preferred_element_type=jnp.float32) + b2_ref[...]


def kernel(questions, question_skill_targets, difficulty_feats, Q_table,
           S_table, W_diff, b_diff, W1, b1, W2, b2):
    B, NT = question_skill_targets.shape
    DF = difficulty_feats.shape[1]
    D = Q_table.shape[1]
    H = W1.shape[1]
    qi = questions.astype(jnp.int32)

    bB = 512
    grid = (B // bB,)
    W1q, W1m, W1a, w1p = W1[0:D], W1[D:2 * D], W1[2 * D:3 * D], W1[3 * D:]
    bd2 = b_diff.reshape(1, D)
    b12 = b1.reshape(1, H)
    b22 = b2.reshape(1, 1)

    full = lambda i: (0, 0)
    rows = lambda i: (i, 0)

    E1, mu, a = pl.pallas_call(
        _tc_a,
        grid=grid,
        in_specs=[
            pl.BlockSpec((bB, NT), rows),
            pl.BlockSpec((bB, DF), rows),
            pl.BlockSpec((NT, D), full),
            pl.BlockSpec((DF, D), full),
            pl.BlockSpec((1, D), full),
            pl.BlockSpec((D, H), full),
            pl.BlockSpec((D, H), full),
            pl.BlockSpec((3, H), full),
            pl.BlockSpec((1, H), full),
        ],
        out_specs=[
            pl.BlockSpec((bB, H), rows),
            pl.BlockSpec((bB, D), rows),
            pl.BlockSpec((bB, D), rows),
        ],
        out_shape=[
            jax.ShapeDtypeStruct((B, H), jnp.float32),
            jax.ShapeDtypeStruct((B, D), jnp.float32),
            jax.ShapeDtypeStruct((B, D), jnp.float32),
        ],
        compiler_params=pltpu.CompilerParams(
            dimension_semantics=("arbitrary",),
        ),
    )(question_skill_targets, difficulty_feats, S_table, W_diff, bd2,
      W1m, W1a, w1p, b12)

    q = _sc_gather(Q_table, qi)

    e, p = pl.pallas_call(
        _tc_b,
        grid=grid,
        in_specs=[
            pl.BlockSpec((bB, D), rows),
            pl.BlockSpec((bB, H), rows),
            pl.BlockSpec((bB, D), rows),
            pl.BlockSpec((bB, D), rows),
            pl.BlockSpec((D, H), full),
            pl.BlockSpec((3, H), full),
            pl.BlockSpec((H, 1), full),
            pl.BlockSpec((1, 1), full),
        ],
        out_specs=[
            pl.BlockSpec((bB, H), rows),
            pl.BlockSpec((bB, 1), rows),
        ],
        out_shape=[
            jax.ShapeDtypeStruct((B, H), jnp.float32),
            jax.ShapeDtypeStruct((B, 1), jnp.float32),
        ],
        compiler_params=pltpu.CompilerParams(
            dimension_semantics=("arbitrary",),
        ),
    )(q, E1, mu, a, W1q, w1p, W2, b22)
    return (e, p)


# speedup vs baseline: 1.0742x; 1.0742x over previous
import functools

import jax
import jax.numpy as jnp
from jax import lax
from jax.experimental import pallas as pl
from jax.experimental.pallas import tpu as pltpu
from jax.experimental.pallas import tpu_sc as plsc


def _sc_gather(table, idx):
    NQ, D = table.shape
    B = idx.shape[0]
    table3 = table.reshape(NQ // 8, 8, D)
    info = plsc.get_sparse_core_info()
    nc, ns, L = info.num_cores, info.num_subcores, info.num_lanes
    nw = nc * ns
    n = B // nw
    K = 16
    nch = n // K
    mesh = plsc.VectorSubcoreMesh(core_axis_name="c", subcore_axis_name="s")

    @functools.partial(
        pl.kernel,
        mesh=mesh,
        out_type=jax.ShapeDtypeStruct((B, D), jnp.float32),
        scratch_types=[
            pltpu.VMEM((n,), jnp.int32),
            pltpu.VMEM((2, K, 8, D), jnp.float32),
            pltpu.VMEM((n, D), jnp.float32),
            pltpu.SemaphoreType.DMA,
            pltpu.SemaphoreType.DMA,
        ],
        compiler_params=pltpu.CompilerParams(needs_layout_passes=False),
    )
    def k(table_hbm, idx_hbm, out_hbm, idx_v, slabs, rows, sem0, sem1):
        wid = lax.axis_index("s") * nc + lax.axis_index("c")
        base = wid * n
        pltpu.sync_copy(idx_hbm.at[pl.ds(base, n)], idx_v)
        sems = (sem0, sem1)

        def fire(c, bank, sem):
            qv = idx_v[pl.ds(c * K, L)]
            slabv = jnp.right_shift(qv, 3)
            for j in range(K):
                pltpu.async_copy(table_hbm.at[slabv[j]],
                                 slabs.at[bank, j], sem)

        def drain_extract(c, bank, sem):
            for j in range(K):
                pltpu.make_async_copy(table_hbm.at[0], slabs.at[bank, j],
                                      sem).wait()
            qv = idx_v[pl.ds(c * K, L)]
            subv = jnp.bitwise_and(qv, 7)
            r0 = c * K
            for j in range(K):
                sub = subv[j]
                for cc in range(D // L):
                    rows[r0 + j, pl.ds(cc * L, L)] = slabs[bank, j, sub,
                                                           pl.ds(cc * L, L)]

        fire(0, 0, sem0)

        def body(h, _):
            c0 = 2 * h
            fire(c0 + 1, 1, sem1)
            drain_extract(c0, 0, sem0)
            nxt = lax.rem(c0 + 2, nch)
            fire(nxt, 0, sem0)
            drain_extract(c0 + 1, 1, sem1)
            return 0

        lax.fori_loop(0, nch // 2, body, 0)
        for j in range(K):
            pltpu.make_async_copy(table_hbm.at[0], slabs.at[0, j],
                                  sem0).wait()
        pltpu.sync_copy(rows, out_hbm.at[pl.ds(base, n)])

    return k(table3, idx)


def _tc_body(t_ref, q_ref, df_ref, S_ref, Wd_ref, bd_ref, W1q_ref, W1m_ref,
             W1a_ref, w1p_ref, b1_ref, W2_ref, b2_ref, e_ref, p_ref):
    mask = (t_ref[...] != 0).astype(jnp.float32)
    cnt = jnp.maximum(jnp.sum(mask, axis=1, keepdims=True), 1.0)
    mu = lax.dot_general(mask, S_ref[...], (((1,), (0,)), ((), ())),
                         preferred_element_type=jnp.float32) / cnt
    q = q_ref[...]
    a = jnp.dot(df_ref[...], Wd_ref[...],
                preferred_element_type=jnp.float32) + bd_ref[...]
    p12 = jnp.sum(q * mu, axis=-1, keepdims=True)
    p13 = jnp.sum(q * a, axis=-1, keepdims=True)
    p23 = jnp.sum(mu * a, axis=-1, keepdims=True)
    z = (jnp.dot(q, W1q_ref[...], preferred_element_type=jnp.float32)
         + jnp.dot(mu, W1m_ref[...], preferred_element_type=jnp.float32)
         + jnp.dot(a, W1a_ref[...], preferred_element_type=jnp.float32)
         + p12 * w1p_ref[0:1, :] + p13 * w1p_ref[1:2, :]
         + p23 * w1p_ref[2:3, :] + b1_ref[...])
    e = jnp.maximum(z, 0.0)
    e_ref[...] = e
    p_ref[...] = jnp.dot(e, W2_ref[...],
                         preferred_element_type=jnp.float32) + b2_ref[...]


def kernel(questions, question_skill_targets, difficulty_feats, Q_table,
           S_table, W_diff, b_diff, W1, b1, W2, b2):
    B, NT = question_skill_targets.shape
    DF = difficulty_feats.shape[1]
    D = Q_table.shape[1]
    H = W1.shape[1]
    qi = questions.astype(jnp.int32)

    bB = 512
    grid = (B // bB,)
    W1q, W1m, W1a, w1p = W1[0:D], W1[D:2 * D], W1[2 * D:3 * D], W1[3 * D:]
    bd2 = b_diff.reshape(1, D)
    b12 = b1.reshape(1, H)
    b22 = b2.reshape(1, 1)

    full = lambda i: (0, 0)
    rows = lambda i: (i, 0)

    q = _sc_gather(Q_table, qi)

    e, p = pl.pallas_call(
        _tc_body,
        grid=grid,
        in_specs=[
            pl.BlockSpec((bB, NT), rows),
            pl.BlockSpec((bB, D), rows),
            pl.BlockSpec((bB, DF), rows),
            pl.BlockSpec((NT, D), full),
            pl.BlockSpec((DF, D), full),
            pl.BlockSpec((1, D), full),
            pl.BlockSpec((D, H), full),
            pl.BlockSpec((D, H), full),
            pl.BlockSpec((D, H), full),
            pl.BlockSpec((3, H), full),
            pl.BlockSpec((1, H), full),
            pl.BlockSpec((H, 1), full),
            pl.BlockSpec((1, 1), full),
        ],
        out_specs=[
            pl.BlockSpec((bB, H), rows),
            pl.BlockSpec((bB, 1), rows),
        ],
        out_shape=[
            jax.ShapeDtypeStruct((B, H), jnp.float32),
            jax.ShapeDtypeStruct((B, 1), jnp.float32),
        ],
        compiler_params=pltpu.CompilerParams(
            dimension_semantics=("arbitrary",),
        ),
    )(question_skill_targets, q, difficulty_feats, S_table, W_diff, bd2,
      W1q, W1m, W1a, w1p, b12, W2, b22)
    return (e, p)


# speedup vs baseline: 1.4282x; 1.3296x over previous
import functools

import jax
import jax.numpy as jnp
from jax import lax
from jax.experimental import pallas as pl
from jax.experimental.pallas import tpu as pltpu
from jax.experimental.pallas import tpu_sc as plsc


def _sc_gather(table, idx):
    NQ, D = table.shape
    B = idx.shape[0]
    tableT = jnp.transpose(table)
    info = plsc.get_sparse_core_info()
    nc, ns, L = info.num_cores, info.num_subcores, info.num_lanes
    nw = nc * ns
    CW = 1024
    NCH = -(-NQ // CW)
    TAIL = NCH - 1
    TW = 640
    t0 = NQ - TW
    tailT = lax.slice(tableT, (0, t0), (D, NQ))
    CAP = 1024
    mesh = plsc.VectorSubcoreMesh(core_axis_name="c", subcore_axis_name="s")

    @functools.partial(
        pl.kernel,
        mesh=mesh,
        out_type=jax.ShapeDtypeStruct((B * D,), jnp.float32),
        scratch_types=[
            pltpu.VMEM((B,), jnp.int32),
            pltpu.VMEM((CAP,), jnp.int32),
            pltpu.VMEM((CAP,), jnp.int32),
            pltpu.VMEM((64,), jnp.int32),
            pltpu.VMEM((64,), jnp.int32),
            pltpu.VMEM((D, CW), jnp.float32),
            pltpu.VMEM((L, D), jnp.float32),
            pltpu.SemaphoreType.DMA,
            pltpu.SemaphoreType.DMA,
        ],
        compiler_params=pltpu.CompilerParams(needs_layout_passes=False),
    )
    def k(tab_hbm, tail_hbm, idx_hbm, out_hbm, idx_v, mq, mi, chq, chi,
          stage, rowbuf, sem, sem2):
        wid = lax.axis_index("s") * nc + lax.axis_index("c")
        pltpu.sync_copy(idx_hbm, idx_v)
        iota = lax.iota(jnp.int32, L)

        def scan(t, off):
            v = idx_v[pl.ds(t * L, L)]
            m = jnp.bitwise_and(jnp.right_shift(v, 10), nw - 1) == wid
            plsc.store_compressed(mq.at[pl.ds(off, L)], v, mask=m)
            plsc.store_compressed(mi.at[pl.ds(off, L)], iota + t * L, mask=m)
            return off + plsc.all_reduce_population_count(m)[0]

        total = lax.fori_loop(0, B // L, scan, 0)

        nmy = 30 + jnp.where(wid <= TAIL % nw, 1, 0)

        def chunk_loop(t, _):
            c_id = wid + nw * t
            base = c_id * CW

            @pl.when(c_id != TAIL)
            def _():
                pltpu.async_copy(tab_hbm.at[:, pl.ds(base, CW)], stage,
                                 sem).wait()

            @pl.when(c_id == TAIL)
            def _():
                pltpu.async_copy(tail_hbm,
                                 stage.at[:, pl.ds(0, TW)], sem).wait()

            qbase = jnp.where(c_id == TAIL, t0, base)

            def scan2(s, off2):
                v = mq[pl.ds(s * L, L)]
                pi = mi[pl.ds(s * L, L)]
                m2 = jnp.logical_and(jnp.right_shift(v, 10) == c_id,
                                     iota + s * L < total)
                plsc.store_compressed(chq.at[pl.ds(off2, L)], v, mask=m2)
                plsc.store_compressed(chi.at[pl.ds(off2, L)], pi, mask=m2)
                return off2 + plsc.all_reduce_population_count(m2)[0]

            cnt2 = lax.fori_loop(0, CAP // L, scan2, 0)

            def egroup(g, _):
                ql = jnp.bitwise_and(chq[pl.ds(g * L, L)] - qbase, CW - 1)
                pos = chi[pl.ds(g * L, L)]
                for kk in range(D):
                    kv = jnp.full((L,), kk, jnp.int32)
                    v = plsc.load_gather(stage, [kv, ql])
                    plsc.store_scatter(rowbuf, [iota, kv], v)
                nv = jnp.minimum(cnt2 - g * L, L)
                for j in range(L):
                    @pl.when(j < nv)
                    def _():
                        dst = pl.multiple_of(pos[j] * D, 8)
                        pltpu.async_copy(rowbuf.at[j],
                                         out_hbm.at[pl.ds(dst, D)], sem2)

                def dr(u, _):
                    pltpu.make_async_copy(rowbuf.at[0],
                                          out_hbm.at[pl.ds(0, D)],
                                          sem2).wait()
                    return 0

                lax.fori_loop(0, nv, dr, 0)
                return 0

            lax.fori_loop(0, (cnt2 + L - 1) // L, egroup, 0)
            return 0

        lax.fori_loop(0, nmy, chunk_loop, 0)

    return k(tableT, tailT, idx).reshape(B, D)


def _tc_body(t_ref, q_ref, df_ref, S_ref, Wd_ref, bd_ref, W1q_ref, W1m_ref,
             W1a_ref, w1p_ref, b1_ref, W2_ref, b2_ref, e_ref, p_ref):
    mask = (t_ref[...] != 0).astype(jnp.float32)
    cnt = jnp.maximum(jnp.sum(mask, axis=1, keepdims=True), 1.0)
    mu = lax.dot_general(mask, S_ref[...], (((1,), (0,)), ((), ())),
                         preferred_element_type=jnp.float32) / cnt
    q = q_ref[...]
    a = jnp.dot(df_ref[...], Wd_ref[...],
                preferred_element_type=jnp.float32) + bd_ref[...]
    p12 = jnp.sum(q * mu, axis=-1, keepdims=True)
    p13 = jnp.sum(q * a, axis=-1, keepdims=True)
    p23 = jnp.sum(mu * a, axis=-1, keepdims=True)
    z = (jnp.dot(q, W1q_ref[...], preferred_element_type=jnp.float32)
         + jnp.dot(mu, W1m_ref[...], preferred_element_type=jnp.float32)
         + jnp.dot(a, W1a_ref[...], preferred_element_type=jnp.float32)
         + p12 * w1p_ref[0:1, :] + p13 * w1p_ref[1:2, :]
         + p23 * w1p_ref[2:3, :] + b1_ref[...])
    e = jnp.maximum(z, 0.0)
    e_ref[...] = e
    p_ref[...] = jnp.dot(e, W2_ref[...],
                         preferred_element_type=jnp.float32) + b2_ref[...]


def kernel(questions, question_skill_targets, difficulty_feats, Q_table,
           S_table, W_diff, b_diff, W1, b1, W2, b2):
    B, NT = question_skill_targets.shape
    DF = difficulty_feats.shape[1]
    D = Q_table.shape[1]
    H = W1.shape[1]
    qi = questions.astype(jnp.int32)

    bB = 512
    grid = (B // bB,)
    W1q, W1m, W1a, w1p = W1[0:D], W1[D:2 * D], W1[2 * D:3 * D], W1[3 * D:]
    bd2 = b_diff.reshape(1, D)
    b12 = b1.reshape(1, H)
    b22 = b2.reshape(1, 1)

    full = lambda i: (0, 0)
    rows = lambda i: (i, 0)

    q = _sc_gather(Q_table, qi)

    e, p = pl.pallas_call(
        _tc_body,
        grid=grid,
        in_specs=[
            pl.BlockSpec((bB, NT), rows),
            pl.BlockSpec((bB, D), rows),
            pl.BlockSpec((bB, DF), rows),
            pl.BlockSpec((NT, D), full),
            pl.BlockSpec((DF, D), full),
            pl.BlockSpec((1, D), full),
            pl.BlockSpec((D, H), full),
            pl.BlockSpec((D, H), full),
            pl.BlockSpec((D, H), full),
            pl.BlockSpec((3, H), full),
            pl.BlockSpec((1, H), full),
            pl.BlockSpec((H, 1), full),
            pl.BlockSpec((1, 1), full),
        ],
        out_specs=[
            pl.BlockSpec((bB, H), rows),
            pl.BlockSpec((bB, 1), rows),
        ],
        out_shape=[
            jax.ShapeDtypeStruct((B, H), jnp.float32),
            jax.ShapeDtypeStruct((B, 1), jnp.float32),
        ],
        compiler_params=pltpu.CompilerParams(
            dimension_semantics=("arbitrary",),
        ),
    )(question_skill_targets, q, difficulty_feats, S_table, W_diff, bd2,
      W1q, W1m, W1a, w1p, b12, W2, b22)
    return (e, p)


# speedup vs baseline: 1.5345x; 1.0744x over previous
import functools

import jax
import jax.numpy as jnp
from jax import lax
from jax.experimental import pallas as pl
from jax.experimental.pallas import tpu as pltpu
from jax.experimental.pallas import tpu_sc as plsc


def _sc_gather(table, idx):
    NQ, D = table.shape
    B = idx.shape[0]
    tableT = jnp.transpose(table)
    info = plsc.get_sparse_core_info()
    nc, ns, L = info.num_cores, info.num_subcores, info.num_lanes
    nw = nc * ns
    CW = 1024
    NCH = -(-NQ // CW)
    TAIL = NCH - 1
    TW = 640
    t0 = NQ - TW
    tailT = lax.slice(tableT, (0, t0), (D, NQ))
    CAP = 1024
    mesh = plsc.VectorSubcoreMesh(core_axis_name="c", subcore_axis_name="s")

    @functools.partial(
        pl.kernel,
        mesh=mesh,
        out_type=jax.ShapeDtypeStruct((B * D,), jnp.float32),
        scratch_types=[
            pltpu.VMEM((B,), jnp.int32),
            pltpu.VMEM((CAP,), jnp.int32),
            pltpu.VMEM((CAP,), jnp.int32),
            pltpu.VMEM((64,), jnp.int32),
            pltpu.VMEM((64,), jnp.int32),
            pltpu.VMEM((D, CW), jnp.float32),
            pltpu.VMEM((L, D), jnp.float32),
            pltpu.SemaphoreType.DMA,
            pltpu.SemaphoreType.DMA,
        ],
        compiler_params=pltpu.CompilerParams(needs_layout_passes=False),
    )
    def k(tab_hbm, tail_hbm, idx_hbm, out_hbm, idx_v, mq, mi, chq, chi,
          stage, rowbuf, sem, sem2):
        wid = lax.axis_index("s") * nc + lax.axis_index("c")
        pltpu.sync_copy(idx_hbm, idx_v)
        iota = lax.iota(jnp.int32, L)

        def scan(t, off):
            v = idx_v[pl.ds(t * L, L)]
            m = jnp.bitwise_and(jnp.right_shift(v, 10), nw - 1) == wid
            plsc.store_compressed(mq.at[pl.ds(off, L)], v, mask=m)
            plsc.store_compressed(mi.at[pl.ds(off, L)], iota + t * L, mask=m)
            return off + plsc.all_reduce_population_count(m)[0]

        total = lax.fori_loop(0, B // L, scan, 0)

        nmy = 30 + jnp.where(wid <= TAIL % nw, 1, 0)

        def chunk_loop(t, _):
            c_id = wid + nw * t
            base = c_id * CW

            @pl.when(c_id != TAIL)
            def _():
                pltpu.async_copy(tab_hbm.at[:, pl.ds(base, CW)], stage, sem)

            @pl.when(c_id == TAIL)
            def _():
                pltpu.async_copy(tail_hbm, stage.at[:, pl.ds(0, TW)], sem)

            qbase = jnp.where(c_id == TAIL, t0, base)

            def scan2(s, off2):
                v = mq[pl.ds(s * L, L)]
                pi = mi[pl.ds(s * L, L)]
                m2 = jnp.logical_and(jnp.right_shift(v, 10) == c_id,
                                     iota + s * L < total)
                plsc.store_compressed(chq.at[pl.ds(off2, L)], v, mask=m2)
                plsc.store_compressed(chi.at[pl.ds(off2, L)], pi, mask=m2)
                return off2 + plsc.all_reduce_population_count(m2)[0]

            cnt2 = lax.fori_loop(0, lax.div(total + L - 1, L), scan2, 0)

            @pl.when(c_id != TAIL)
            def _():
                pltpu.make_async_copy(tab_hbm.at[:, pl.ds(base, CW)], stage,
                                      sem).wait()

            @pl.when(c_id == TAIL)
            def _():
                pltpu.make_async_copy(tail_hbm, stage.at[:, pl.ds(0, TW)],
                                      sem).wait()

            def egroup(g, _):
                ql = jnp.bitwise_and(chq[pl.ds(g * L, L)] - qbase, CW - 1)
                pos = chi[pl.ds(g * L, L)]
                for kk in range(D):
                    kv = jnp.full((L,), kk, jnp.int32)
                    v = plsc.load_gather(stage, [kv, ql])
                    plsc.store_scatter(rowbuf, [iota, kv], v)
                nv = jnp.minimum(cnt2 - g * L, L)
                for j in range(L):
                    @pl.when(j < nv)
                    def _():
                        dst = pl.multiple_of(pos[j] * D, 8)
                        pltpu.async_copy(rowbuf.at[j],
                                         out_hbm.at[pl.ds(dst, D)], sem2)

                def dr(u, _):
                    pltpu.make_async_copy(rowbuf.at[0],
                                          out_hbm.at[pl.ds(0, D)],
                                          sem2).wait()
                    return 0

                lax.fori_loop(0, nv, dr, 0)
                return 0

            lax.fori_loop(0, (cnt2 + L - 1) // L, egroup, 0)
            return 0

        lax.fori_loop(0, nmy, chunk_loop, 0)

    return k(tableT, tailT, idx).reshape(B, D)


def _tc_body(t_ref, q_ref, df_ref, S_ref, Wd_ref, bd_ref, W1q_ref, W1m_ref,
             W1a_ref, w1p_ref, b1_ref, W2_ref, b2_ref, e_ref, p_ref):
    mask = (t_ref[...] != 0).astype(jnp.float32)
    cnt = jnp.maximum(jnp.sum(mask, axis=1, keepdims=True), 1.0)
    mu = lax.dot_general(mask, S_ref[...], (((1,), (0,)), ((), ())),
                         preferred_element_type=jnp.float32) / cnt
    q = q_ref[...]
    a = jnp.dot(df_ref[...], Wd_ref[...],
                preferred_element_type=jnp.float32) + bd_ref[...]
    p12 = jnp.sum(q * mu, axis=-1, keepdims=True)
    p13 = jnp.sum(q * a, axis=-1, keepdims=True)
    p23 = jnp.sum(mu * a, axis=-1, keepdims=True)
    z = (jnp.dot(q, W1q_ref[...], preferred_element_type=jnp.float32)
         + jnp.dot(mu, W1m_ref[...], preferred_element_type=jnp.float32)
         + jnp.dot(a, W1a_ref[...], preferred_element_type=jnp.float32)
         + p12 * w1p_ref[0:1, :] + p13 * w1p_ref[1:2, :]
         + p23 * w1p_ref[2:3, :] + b1_ref[...])
    e = jnp.maximum(z, 0.0)
    e_ref[...] = e
    p_ref[...] = jnp.dot(e, W2_ref[...],
                         preferred_element_type=jnp.float32) + b2_ref[...]


def kernel(questions, question_skill_targets, difficulty_feats, Q_table,
           S_table, W_diff, b_diff, W1, b1, W2, b2):
    B, NT = question_skill_targets.shape
    DF = difficulty_feats.shape[1]
    D = Q_table.shape[1]
    H = W1.shape[1]
    qi = questions.astype(jnp.int32)

    bB = 512
    grid = (B // bB,)
    W1q, W1m, W1a, w1p = W1[0:D], W1[D:2 * D], W1[2 * D:3 * D], W1[3 * D:]
    bd2 = b_diff.reshape(1, D)
    b12 = b1.reshape(1, H)
    b22 = b2.reshape(1, 1)

    full = lambda i: (0, 0)
    rows = lambda i: (i, 0)

    q = _sc_gather(Q_table, qi)

    e, p = pl.pallas_call(
        _tc_body,
        grid=grid,
        in_specs=[
            pl.BlockSpec((bB, NT), rows),
            pl.BlockSpec((bB, D), rows),
            pl.BlockSpec((bB, DF), rows),
            pl.BlockSpec((NT, D), full),
            pl.BlockSpec((DF, D), full),
            pl.BlockSpec((1, D), full),
            pl.BlockSpec((D, H), full),
            pl.BlockSpec((D, H), full),
            pl.BlockSpec((D, H), full),
            pl.BlockSpec((3, H), full),
            pl.BlockSpec((1, H), full),
            pl.BlockSpec((H, 1), full),
            pl.BlockSpec((1, 1), full),
        ],
        out_specs=[
            pl.BlockSpec((bB, H), rows),
            pl.BlockSpec((bB, 1), rows),
        ],
        out_shape=[
            jax.ShapeDtypeStruct((B, H), jnp.float32),
            jax.ShapeDtypeStruct((B, 1), jnp.float32),
        ],
        compiler_params=pltpu.CompilerParams(
            dimension_semantics=("arbitrary",),
        ),
    )(question_skill_targets, q, difficulty_feats, S_table, W_diff, bd2,
      W1q, W1m, W1a, w1p, b12, W2, b22)
    return (e, p)


# speedup vs baseline: 1.5863x; 1.0338x over previous
import functools

import jax
import jax.numpy as jnp
from jax import lax
from jax.experimental import pallas as pl
from jax.experimental.pallas import tpu as pltpu
from jax.experimental.pallas import tpu_sc as plsc


def _sc_gather(table, idx):
    NQ, D = table.shape
    B = idx.shape[0]
    tableT = jnp.transpose(table)
    info = plsc.get_sparse_core_info()
    nc, ns, L = info.num_cores, info.num_subcores, info.num_lanes
    nw = nc * ns
    CW = 1024
    NCH = -(-NQ // CW)
    TAIL = NCH - 1
    TW = 640
    t0 = NQ - TW
    tailT = lax.slice(tableT, (0, t0), (D, NQ))
    CAP = 1024
    mesh = plsc.VectorSubcoreMesh(core_axis_name="c", subcore_axis_name="s")

    @functools.partial(
        pl.kernel,
        mesh=mesh,
        out_type=jax.ShapeDtypeStruct((B * D,), jnp.float32),
        scratch_types=[
            pltpu.VMEM((B,), jnp.int32),
            pltpu.VMEM((CAP,), jnp.int32),
            pltpu.VMEM((CAP,), jnp.int32),
            pltpu.VMEM((64,), jnp.int32),
            pltpu.VMEM((64,), jnp.int32),
            pltpu.VMEM((D, CW), jnp.float32),
            pltpu.VMEM((L, D), jnp.float32),
            pltpu.SemaphoreType.DMA,
            pltpu.SemaphoreType.DMA,
        ],
        compiler_params=pltpu.CompilerParams(needs_layout_passes=False),
    )
    def k(tab_hbm, tail_hbm, idx_hbm, out_hbm, idx_v, mq, mi, chq, chi,
          stage, rowbuf, sem, sem2):
        wid = lax.axis_index("s") * nc + lax.axis_index("c")
        pltpu.sync_copy(idx_hbm, idx_v)
        iota = lax.iota(jnp.int32, L)

        def scan(t, off):
            v = idx_v[pl.ds(t * L, L)]
            m = jnp.bitwise_and(jnp.right_shift(v, 10), nw - 1) == wid
            plsc.store_compressed(mq.at[pl.ds(off, L)], v, mask=m)
            plsc.store_compressed(mi.at[pl.ds(off, L)], iota + t * L, mask=m)
            return off + plsc.all_reduce_population_count(m)[0]

        total = lax.fori_loop(0, B // L, scan, 0)

        nmy = 30 + jnp.where(wid <= TAIL % nw, 1, 0)

        def chunk_loop(t, _):
            c_id = wid + nw * t
            base = c_id * CW

            @pl.when(c_id != TAIL)
            def _():
                pltpu.async_copy(tab_hbm.at[:, pl.ds(base, CW)], stage, sem)

            @pl.when(c_id == TAIL)
            def _():
                pltpu.async_copy(tail_hbm, stage.at[:, pl.ds(0, TW)], sem)

            qbase = jnp.where(c_id == TAIL, t0, base)

            def scan2(s, off2):
                v = mq[pl.ds(s * L, L)]
                pi = mi[pl.ds(s * L, L)]
                m2 = jnp.logical_and(jnp.right_shift(v, 10) == c_id,
                                     iota + s * L < total)
                plsc.store_compressed(chq.at[pl.ds(off2, L)], v, mask=m2)
                plsc.store_compressed(chi.at[pl.ds(off2, L)], pi, mask=m2)
                return off2 + plsc.all_reduce_population_count(m2)[0]

            cnt2 = lax.fori_loop(0, lax.div(total + L - 1, L), scan2, 0)

            @pl.when(c_id != TAIL)
            def _():
                pltpu.make_async_copy(tab_hbm.at[:, pl.ds(base, CW)], stage,
                                      sem).wait()

            @pl.when(c_id == TAIL)
            def _():
                pltpu.make_async_copy(tail_hbm, stage.at[:, pl.ds(0, TW)],
                                      sem).wait()

            def egroup(g, _):
                ql = jnp.bitwise_and(chq[pl.ds(g * L, L)] - qbase, CW - 1)
                pos = chi[pl.ds(g * L, L)]
                for kk in range(D):
                    kv = jnp.full((L,), kk, jnp.int32)
                    v = plsc.load_gather(stage, [kv, ql])
                    plsc.store_scatter(rowbuf, [iota, kv], v)
                nv = jnp.minimum(cnt2 - g * L, L)
                for j in range(L):
                    @pl.when(j < nv)
                    def _():
                        dst = pl.multiple_of(pos[j] * D, 8)
                        pltpu.async_copy(rowbuf.at[j],
                                         out_hbm.at[pl.ds(dst, D)], sem2)

                def dr(u, _):
                    pltpu.make_async_copy(rowbuf.at[0],
                                          out_hbm.at[pl.ds(0, D)],
                                          sem2).wait()
                    return 0

                lax.fori_loop(0, nv, dr, 0)
                return 0

            lax.fori_loop(0, (cnt2 + L - 1) // L, egroup, 0)
            return 0

        lax.fori_loop(0, nmy, chunk_loop, 0)

    return k(tableT, tailT, idx).reshape(B, D)


def _tc_a(t_ref, df_ref, S_ref, Wd_ref, bd_ref, W1m_ref, W1a_ref, w1p_ref,
          b1_ref, E1_ref, mu_ref, a_ref):
    mask = (t_ref[...] != 0).astype(jnp.float32)
    cnt = jnp.maximum(jnp.sum(mask, axis=1, keepdims=True), 1.0)
    mu = lax.dot_general(mask, S_ref[...], (((1,), (0,)), ((), ())),
                         preferred_element_type=jnp.float32) / cnt
    a = jnp.dot(df_ref[...], Wd_ref[...],
                preferred_element_type=jnp.float32) + bd_ref[...]
    p23 = jnp.sum(mu * a, axis=-1, keepdims=True)
    E1 = (jnp.dot(mu, W1m_ref[...], preferred_element_type=jnp.float32)
          + jnp.dot(a, W1a_ref[...], preferred_element_type=jnp.float32)
          + p23 * w1p_ref[2:3, :] + b1_ref[...])
    E1_ref[...] = E1
    mu_ref[...] = mu
    a_ref[...] = a


def _tc_b(q_ref, E1_ref, mu_ref, a_ref, W1q_ref, w1p_ref, W2_ref, b2_ref,
          e_ref, p_ref):
    q = q_ref[...]
    mu = mu_ref[...]
    a = a_ref[...]
    p12 = jnp.sum(q * mu, axis=-1, keepdims=True)
    p13 = jnp.sum(q * a, axis=-1, keepdims=True)
    z = (E1_ref[...] + jnp.dot(q, W1q_ref[...],
                               preferred_element_type=jnp.float32)
         + p12 * w1p_ref[0:1, :] + p13 * w1p_ref[1:2, :])
    e = jnp.maximum(z, 0.0)
    e_ref[...] = e
    p_ref[...] = jnp.dot(e, W2_ref[...],
                         preferred_element_type=jnp.float32) + b2_ref[...]


def kernel(questions, question_skill_targets, difficulty_feats, Q_table,
           S_table, W_diff, b_diff, W1, b1, W2, b2):
    B, NT = question_skill_targets.shape
    DF = difficulty_feats.shape[1]
    D = Q_table.shape[1]
    H = W1.shape[1]
    qi = questions.astype(jnp.int32)

    bB = 512
    grid = (B // bB,)
    W1q, W1m, W1a, w1p = W1[0:D], W1[D:2 * D], W1[2 * D:3 * D], W1[3 * D:]
    bd2 = b_diff.reshape(1, D)
    b12 = b1.reshape(1, H)
    b22 = b2.reshape(1, 1)

    full = lambda i: (0, 0)
    rows = lambda i: (i, 0)

    E1, mu, a = pl.pallas_call(
        _tc_a,
        grid=grid,
        in_specs=[
            pl.BlockSpec((bB, NT), rows),
            pl.BlockSpec((bB, DF), rows),
            pl.BlockSpec((NT, D), full),
            pl.BlockSpec((DF, D), full),
            pl.BlockSpec((1, D), full),
            pl.BlockSpec((D, H), full),
            pl.BlockSpec((D, H), full),
            pl.BlockSpec((3, H), full),
            pl.BlockSpec((1, H), full),
        ],
        out_specs=[
            pl.BlockSpec((bB, H), rows),
            pl.BlockSpec((bB, D), rows),
            pl.BlockSpec((bB, D), rows),
        ],
        out_shape=[
            jax.ShapeDtypeStruct((B, H), jnp.float32),
            jax.ShapeDtypeStruct((B, D), jnp.float32),
            jax.ShapeDtypeStruct((B, D), jnp.float32),
        ],
        compiler_params=pltpu.CompilerParams(
            dimension_semantics=("arbitrary",),
        ),
    )(question_skill_targets, difficulty_feats, S_table, W_diff, bd2,
      W1m, W1a, w1p, b12)

    q = _sc_gather(Q_table, qi)

    e, p = pl.pallas_call(
        _tc_b,
        grid=grid,
        in_specs=[
            pl.BlockSpec((bB, D), rows),
            pl.BlockSpec((bB, H), rows),
            pl.BlockSpec((bB, D), rows),
            pl.BlockSpec((bB, D), rows),
            pl.BlockSpec((D, H), full),
            pl.BlockSpec((3, H), full),
            pl.BlockSpec((H, 1), full),
            pl.BlockSpec((1, 1), full),
        ],
        out_specs=[
            pl.BlockSpec((bB, H), rows),
            pl.BlockSpec((bB, 1), rows),
        ],
        out_shape=[
            jax.ShapeDtypeStruct((B, H), jnp.float32),
            jax.ShapeDtypeStruct((B, 1), jnp.float32),
        ],
        compiler_params=pltpu.CompilerParams(
            dimension_semantics=("arbitrary",),
        ),
    )(q, E1, mu, a, W1q, w1p, W2, b22)
    return (e, p)
